# baseline (speedup 1.0000x reference)
import functools

import jax
import jax.numpy as jnp
from jax import lax
from jax.experimental import pallas as pl
from jax.experimental.pallas import tpu as pltpu
from jax.experimental.pallas import tpu_sc as plsc


H = 12
E = 8
TILE = 512
BC = 512


def _ln(y, g, b):
    m = jnp.mean(y, axis=1, keepdims=True)
    v = jnp.mean((y - m) ** 2, axis=1, keepdims=True)
    return (y - m) * lax.rsqrt(v + 1e-5) * g + b


def _abc_kernel(x2_ref, wqkv_ref, bqkv_ref, wo_ref, bo_ref, g1_ref, b1_ref,
                rw_ref, rb_ref,
                x_ref, gates_ref, pos0_ref, pos1_ref, eid_ref, act_ref,
                qkv_s, ctx_s, topi_s, rank_s, carry_ref,
                *, T, scale, dh, nt, nc):
    i = pl.program_id(0)
    D = H * dh

    @pl.when(i < nc)
    def _():
        rows = x2_ref[...]
        qkv_s[pl.ds(i * BC, BC), :] = (
            lax.dot_general(rows, wqkv_ref[...], (((1,), (1,)), ((), ())),
                            preferred_element_type=jnp.float32)
            + bqkv_ref[...]
        )

    @pl.when(jnp.logical_and(i >= nc, i < 2 * nc))
    def _():
        j = i - nc
        q_all = qkv_s[pl.ds(j * BC, BC), :]
        outs = []
        for h in range(H):
            qh = q_all[:, h * dh:(h + 1) * dh]
            kh = qkv_s[:, D + h * dh:D + (h + 1) * dh]
            vh = qkv_s[:, 2 * D + h * dh:2 * D + (h + 1) * dh]
            s = lax.dot_general(
                qh, kh, (((1,), (1,)), ((), ())),
                preferred_element_type=jnp.float32) * scale
            m = jnp.max(s, axis=1, keepdims=True)
            p = jnp.exp(s - m)
            l = jnp.sum(p, axis=1, keepdims=True)
            outs.append(jnp.dot(p / l, vh, preferred_element_type=jnp.float32))
        ctx_s[pl.ds(j * BC, BC), :] = jnp.concatenate(outs, axis=1)

    @pl.when(i >= 2 * nc)
    def _():
        c = i - 2 * nc

        @pl.when(c == 0)
        def _():
            carry_ref[...] = jnp.zeros_like(carry_ref)

        attn = (
            lax.dot_general(ctx_s[pl.ds(c * BC, BC), :], wo_ref[...],
                            (((1,), (1,)), ((), ())),
                            preferred_element_type=jnp.float32)
            + bo_ref[...]
        )
        x = _ln(x2_ref[...] + attn, g1_ref[...], b1_ref[...])
        x_ref[...] = x
        logits = (
            jnp.dot(x, rw_ref[...], preferred_element_type=jnp.float32)
            + rb_ref[...]
        )
        ii = lax.broadcasted_iota(jnp.int32, (BC, E), 1)
        m1 = jnp.max(logits, axis=1, keepdims=True)
        i1 = jnp.min(jnp.where(logits == m1, ii, E), axis=1, keepdims=True)
        masked = jnp.where(ii == i1, -jnp.inf, logits)
        m2 = jnp.max(masked, axis=1, keepdims=True)
        i2 = jnp.min(jnp.where(masked == m2, ii, E), axis=1, keepdims=True)
        gate1 = 1.0 / (1.0 + jnp.exp(m2 - m1))
        gates_ref[...] = jnp.concatenate([gate1, 1.0 - gate1], axis=1)
        topi_s[pl.ds(c * BC, BC), :] = jnp.concatenate([i1, i2], axis=1)

        rr = lax.broadcasted_iota(jnp.int32, (BC, BC), 0)
        cc = lax.broadcasted_iota(jnp.int32, (BC, BC), 1)
        tril = jnp.where(rr > cc, 1.0, 0.0)
        oh1 = jnp.where(ii == i1, 1.0, 0.0)
        oh2 = jnp.where(ii == i2, 1.0, 0.0)
        carry = carry_ref[...]
        t1 = jnp.dot(tril, oh1, preferred_element_type=jnp.float32)
        rank1 = jnp.sum((t1 + carry) * oh1, axis=1, keepdims=True)
        carry = carry + jnp.sum(oh1, axis=0, keepdims=True)
        t2 = jnp.dot(tril, oh2, preferred_element_type=jnp.float32)
        rank2 = jnp.sum((t2 + carry) * oh2, axis=1, keepdims=True)
        carry = carry + jnp.sum(oh2, axis=0, keepdims=True)
        carry_ref[...] = carry
        rank_s[pl.ds(c * BC, BC), :] = jnp.concatenate(
            [rank1, rank2], axis=1).astype(jnp.int32)

        @pl.when(c == nc - 1)
        def _():
            counts = carry
            padded = jnp.floor((counts + (TILE - 1)) / TILE) * TILE
            utri = jnp.where(
                lax.broadcasted_iota(jnp.int32, (E, E), 0)
                < lax.broadcasted_iota(jnp.int32, (E, E), 1),
                1.0, 0.0)
            off = jnp.dot(padded, utri, preferred_element_type=jnp.float32)
            topi = topi_s[...]
            rank = rank_s[...]
            off_sel = jnp.zeros((T, 2), jnp.float32)
            for e in range(E):
                off_sel = jnp.where(topi == e, off[:, e:e + 1], off_sel)
            pos = off_sel.astype(jnp.int32) + rank
            pos0_ref[...] = pos[:, 0:1]
            pos1_ref[...] = pos[:, 1:2]

            starts = off / TILE
            total = (off[:, E - 1:E] + padded[:, E - 1:E]) / TILE
            tt = lax.broadcasted_iota(jnp.int32, (1, nt), 1).astype(jnp.float32)
            tcl = jnp.minimum(tt, total - 1.0)
            eid = jnp.zeros((1, nt), jnp.float32)
            for e in range(1, E):
                eid = eid + jnp.where(tcl >= starts[:, e:e + 1], 1.0, 0.0)
            eid_ref[...] = eid.astype(jnp.int32)
            act_ref[...] = jnp.where(tt < total, 1, 0).astype(jnp.int32)


def _moe_kernel(eid_ref, act_ref, xs_ref, w1_ref, b1_ref, w2_ref, b2_ref,
                ys_ref):
    t = pl.program_id(0)

    @pl.when(act_ref[t] == 1)
    def _():
        xsb = xs_ref[...].astype(jnp.bfloat16)
        h = jnp.maximum(
            jnp.dot(xsb, w1_ref[0], preferred_element_type=jnp.float32)
            + b1_ref[0],
            0.0,
        ).astype(jnp.bfloat16)
        ys_ref[...] = (
            jnp.dot(h, w2_ref[0], preferred_element_type=jnp.float32)
            + b2_ref[0]
        )


def _final_kernel(x_ref, c0_ref, c1_ref, gates_ref, g2_ref, b2_ref, o_ref):
    gates = gates_ref[...]
    moe = gates[:, 0:1] * c0_ref[...] + gates[:, 1:2] * c1_ref[...]
    o_ref[...] = _ln(x_ref[...] + moe, g2_ref[...], b2_ref[...])


def _dispatch_sc(x, pos0, pos1, cap):
    T, D = x.shape
    info = plsc.get_sparse_core_info()
    nw = info.num_cores * info.num_subcores
    ch = T // nw
    mesh = plsc.VectorSubcoreMesh(core_axis_name="c", subcore_axis_name="s")

    @functools.partial(
        pl.kernel, mesh=mesh,
        out_type=jax.ShapeDtypeStruct((cap, D), jnp.float32),
        scratch_types=[
            pltpu.VMEM((ch,), jnp.int32),
            pltpu.VMEM((ch,), jnp.int32),
            pltpu.VMEM((ch, D), jnp.float32),
            pltpu.SemaphoreType.DMA,
        ],
    )
    def k(x_hbm, p0_hbm, p1_hbm, xs_hbm, i0_v, i1_v, rows_v, sem):
        wid = lax.axis_index("s") * info.num_cores + lax.axis_index("c")
        base = wid * ch
        pltpu.sync_copy(x_hbm.at[pl.ds(base, ch)], rows_v)
        pltpu.sync_copy(p0_hbm.at[pl.ds(base, ch)], i0_v)
        pltpu.sync_copy(p1_hbm.at[pl.ds(base, ch)], i1_v)
        pltpu.async_copy(rows_v, xs_hbm.at[i0_v], sem).wait()
        pltpu.async_copy(rows_v, xs_hbm.at[i1_v], sem).wait()

    return k(x, pos0, pos1)


def _combine_sc(ys, pos0, pos1, T):
    D = ys.shape[1]
    info = plsc.get_sparse_core_info()
    nw = info.num_cores * info.num_subcores
    ch = T // nw
    mesh = plsc.VectorSubcoreMesh(core_axis_name="c", subcore_axis_name="s")

    @functools.partial(
        pl.kernel, mesh=mesh,
        out_type=[
            jax.ShapeDtypeStruct((T, D), jnp.float32),
            jax.ShapeDtypeStruct((T, D), jnp.float32),
        ],
        scratch_types=[
            pltpu.VMEM((ch,), jnp.int32),
            pltpu.VMEM((ch, D), jnp.float32),
            pltpu.SemaphoreType.DMA,
        ],
    )
    def k(ys_hbm, p0_hbm, p1_hbm, c0_hbm, c1_hbm, idx_v, rows_v, sem):
        wid = lax.axis_index("s") * info.num_cores + lax.axis_index("c")
        base = wid * ch
        pltpu.sync_copy(p0_hbm.at[pl.ds(base, ch)], idx_v)
        pltpu.async_copy(ys_hbm.at[idx_v], rows_v, sem).wait()
        pltpu.sync_copy(rows_v, c0_hbm.at[pl.ds(base, ch)])
        pltpu.sync_copy(p1_hbm.at[pl.ds(base, ch)], idx_v)
        pltpu.async_copy(ys_hbm.at[idx_v], rows_v, sem).wait()
        pltpu.sync_copy(rows_v, c1_hbm.at[pl.ds(base, ch)])

    return k(ys, pos0, pos1)


def kernel(src, in_proj_w, in_proj_b, out_proj_w, out_proj_b, ln1_g, ln1_b,
           ln2_g, ln2_b, router_w, router_b, w1, b1, w2, b2):
    Bq, T, D = src.shape
    dh = D // H
    F = w1.shape[2]
    x2 = src.reshape(T, D)

    nc = T // BC
    cap = 2 * T + (E - 1) * TILE
    nt = cap // TILE
    def xmap(i):
        return (jnp.maximum(i - 2 * nc, 0), 0)

    x, gates, pos0, pos1, eid, act = pl.pallas_call(
        functools.partial(_abc_kernel, T=T, scale=1.0 / (dh ** 0.5), dh=dh,
                          nt=nt, nc=nc),
        grid=(3 * nc,),
        in_specs=[
            pl.BlockSpec((BC, D), lambda i: (
                jnp.where(i < nc, i, jnp.maximum(i - 2 * nc, 0)), 0)),
            pl.BlockSpec((3 * D, D), lambda i: (0, 0)),
            pl.BlockSpec((1, 3 * D), lambda i: (0, 0)),
            pl.BlockSpec((D, D), lambda i: (0, 0)),
            pl.BlockSpec((1, D), lambda i: (0, 0)),
            pl.BlockSpec((1, D), lambda i: (0, 0)),
            pl.BlockSpec((1, D), lambda i: (0, 0)),
            pl.BlockSpec((D, E), lambda i: (0, 0)),
            pl.BlockSpec((1, E), lambda i: (0, 0)),
        ],
        out_specs=[
            pl.BlockSpec((BC, D), xmap),
            pl.BlockSpec((BC, 2), xmap),
            pl.BlockSpec((T, 1), lambda i: (0, 0)),
            pl.BlockSpec((T, 1), lambda i: (0, 0)),
            pl.BlockSpec((1, nt), lambda i: (0, 0)),
            pl.BlockSpec((1, nt), lambda i: (0, 0)),
        ],
        out_shape=[
            jax.ShapeDtypeStruct((T, D), jnp.float32),
            jax.ShapeDtypeStruct((T, 2), jnp.float32),
            jax.ShapeDtypeStruct((T, 1), jnp.int32),
            jax.ShapeDtypeStruct((T, 1), jnp.int32),
            jax.ShapeDtypeStruct((1, nt), jnp.int32),
            jax.ShapeDtypeStruct((1, nt), jnp.int32),
        ],
        scratch_shapes=[
            pltpu.VMEM((T, 3 * D), jnp.float32),
            pltpu.VMEM((T, D), jnp.float32),
            pltpu.VMEM((T, 2), jnp.int32),
            pltpu.VMEM((T, 2), jnp.int32),
            pltpu.VMEM((1, E), jnp.float32),
        ],
    )(x2, in_proj_w, in_proj_b.reshape(1, 3 * D),
      out_proj_w, out_proj_b.reshape(1, D),
      ln1_g.reshape(1, D), ln1_b.reshape(1, D),
      router_w, router_b.reshape(1, E))

    w1b = w1.astype(jnp.bfloat16)
    w2b = w2.astype(jnp.bfloat16)

    xs = _dispatch_sc(x, pos0.reshape(T), pos1.reshape(T), cap)

    grid_spec = pltpu.PrefetchScalarGridSpec(
        num_scalar_prefetch=2,
        grid=(nt,),
        in_specs=[
            pl.BlockSpec((TILE, D), lambda t, eid, act: (t, 0)),
            pl.BlockSpec((1, D, F), lambda t, eid, act: (eid[t], 0, 0)),
            pl.BlockSpec((1, 1, F), lambda t, eid, act: (eid[t], 0, 0)),
            pl.BlockSpec((1, F, D), lambda t, eid, act: (eid[t], 0, 0)),
            pl.BlockSpec((1, 1, D), lambda t, eid, act: (eid[t], 0, 0)),
        ],
        out_specs=pl.BlockSpec((TILE, D), lambda t, eid, act: (t, 0)),
    )
    ys = pl.pallas_call(
        _moe_kernel,
        grid_spec=grid_spec,
        out_shape=jax.ShapeDtypeStruct((cap, D), jnp.float32),
    )(eid.reshape(nt), act.reshape(nt), xs,
      w1b, b1.reshape(E, 1, F),
      w2b, b2.reshape(E, 1, D))

    c0, c1 = _combine_sc(ys, pos0.reshape(T), pos1.reshape(T), T)

    bf = min(512, T)
    out = pl.pallas_call(
        _final_kernel,
        grid=(T // bf,),
        in_specs=[
            pl.BlockSpec((bf, D), lambda i: (i, 0)),
            pl.BlockSpec((bf, D), lambda i: (i, 0)),
            pl.BlockSpec((bf, D), lambda i: (i, 0)),
            pl.BlockSpec((bf, 2), lambda i: (i, 0)),
            pl.BlockSpec((1, D), lambda i: (0, 0)),
            pl.BlockSpec((1, D), lambda i: (0, 0)),
        ],
        out_specs=pl.BlockSpec((bf, D), lambda i: (i, 0)),
        out_shape=jax.ShapeDtypeStruct((T, D), jnp.float32),
    )(x, c0, c1, gates, ln2_g.reshape(1, D), ln2_b.reshape(1, D))

    return out.reshape(Bq, T, D)


# speedup vs baseline: 1.1241x; 1.1241x over previous
import functools

import jax
import jax.numpy as jnp
from jax import lax
from jax.experimental import pallas as pl
from jax.experimental.pallas import tpu as pltpu
from jax.experimental.pallas import tpu_sc as plsc


H = 12
E = 8
TILE = 512
BC = 512


def _ln(y, g, b):
    m = jnp.mean(y, axis=1, keepdims=True)
    v = jnp.mean((y - m) ** 2, axis=1, keepdims=True)
    return (y - m) * lax.rsqrt(v + 1e-5) * g + b


def _abc_kernel(x2_ref, wqkv_ref, bqkv_ref, wo_ref, bo_ref, g1_ref, b1_ref,
                rw_ref, rb_ref,
                x_ref, gates_ref, pos0_ref, pos1_ref, eid_ref, act_ref,
                qkv_s, ctx_s, topi_s, rank_s, carry_ref,
                *, T, scale, dh, nt, nc):
    i = pl.program_id(0)
    D = H * dh

    @pl.when(i < nc)
    def _():
        rows = x2_ref[...]
        qkv_s[pl.ds(i * BC, BC), :] = (
            lax.dot_general(rows, wqkv_ref[...], (((1,), (1,)), ((), ())),
                            preferred_element_type=jnp.float32)
            + bqkv_ref[...]
        )

    @pl.when(jnp.logical_and(i >= nc, i < 2 * nc))
    def _():
        j = i - nc
        q_all = qkv_s[pl.ds(j * BC, BC), :]
        outs = []
        for h in range(H):
            qh = q_all[:, h * dh:(h + 1) * dh]
            kh = qkv_s[:, D + h * dh:D + (h + 1) * dh]
            vh = qkv_s[:, 2 * D + h * dh:2 * D + (h + 1) * dh]
            s = lax.dot_general(
                qh, kh, (((1,), (1,)), ((), ())),
                preferred_element_type=jnp.float32) * scale
            m = jnp.max(s, axis=1, keepdims=True)
            p = jnp.exp(s - m)
            l = jnp.sum(p, axis=1, keepdims=True)
            outs.append(jnp.dot(p / l, vh, preferred_element_type=jnp.float32))
        ctx_s[pl.ds(j * BC, BC), :] = jnp.concatenate(outs, axis=1)

    @pl.when(i >= 2 * nc)
    def _():
        c = i - 2 * nc

        @pl.when(c == 0)
        def _():
            carry_ref[...] = jnp.zeros_like(carry_ref)

        attn = (
            lax.dot_general(ctx_s[pl.ds(c * BC, BC), :], wo_ref[...],
                            (((1,), (1,)), ((), ())),
                            preferred_element_type=jnp.float32)
            + bo_ref[...]
        )
        x = _ln(x2_ref[...] + attn, g1_ref[...], b1_ref[...])
        x_ref[...] = x
        logits = (
            jnp.dot(x, rw_ref[...], preferred_element_type=jnp.float32)
            + rb_ref[...]
        )
        ii = lax.broadcasted_iota(jnp.int32, (BC, E), 1)
        m1 = jnp.max(logits, axis=1, keepdims=True)
        i1 = jnp.min(jnp.where(logits == m1, ii, E), axis=1, keepdims=True)
        masked = jnp.where(ii == i1, -jnp.inf, logits)
        m2 = jnp.max(masked, axis=1, keepdims=True)
        i2 = jnp.min(jnp.where(masked == m2, ii, E), axis=1, keepdims=True)
        gate1 = 1.0 / (1.0 + jnp.exp(m2 - m1))
        gates_ref[...] = jnp.concatenate([gate1, 1.0 - gate1], axis=1)
        topi_s[pl.ds(c * BC, BC), :] = jnp.concatenate([i1, i2], axis=1)

        rr = lax.broadcasted_iota(jnp.int32, (BC, BC), 0)
        cc = lax.broadcasted_iota(jnp.int32, (BC, BC), 1)
        tril = jnp.where(rr > cc, 1.0, 0.0)
        oh1 = jnp.where(ii == i1, 1.0, 0.0)
        oh2 = jnp.where(ii == i2, 1.0, 0.0)
        carry = carry_ref[...]
        t1 = jnp.dot(tril, oh1, preferred_element_type=jnp.float32)
        rank1 = jnp.sum((t1 + carry) * oh1, axis=1, keepdims=True)
        carry = carry + jnp.sum(oh1, axis=0, keepdims=True)
        t2 = jnp.dot(tril, oh2, preferred_element_type=jnp.float32)
        rank2 = jnp.sum((t2 + carry) * oh2, axis=1, keepdims=True)
        carry = carry + jnp.sum(oh2, axis=0, keepdims=True)
        carry_ref[...] = carry
        rank_s[pl.ds(c * BC, BC), :] = jnp.concatenate(
            [rank1, rank2], axis=1).astype(jnp.int32)

        @pl.when(c == nc - 1)
        def _():
            counts = carry
            padded = jnp.floor((counts + (TILE - 1)) / TILE) * TILE
            utri = jnp.where(
                lax.broadcasted_iota(jnp.int32, (E, E), 0)
                < lax.broadcasted_iota(jnp.int32, (E, E), 1),
                1.0, 0.0)
            off = jnp.dot(padded, utri, preferred_element_type=jnp.float32)
            topi = topi_s[...]
            rank = rank_s[...]
            off_sel = jnp.zeros((T, 2), jnp.float32)
            for e in range(E):
                off_sel = jnp.where(topi == e, off[:, e:e + 1], off_sel)
            pos = off_sel.astype(jnp.int32) + rank
            pos0_ref[...] = pos[:, 0:1]
            pos1_ref[...] = pos[:, 1:2]

            starts = off / TILE
            total = (off[:, E - 1:E] + padded[:, E - 1:E]) / TILE
            tt = lax.broadcasted_iota(jnp.int32, (1, nt), 1).astype(jnp.float32)
            tcl = jnp.minimum(tt, total - 1.0)
            eid = jnp.zeros((1, nt), jnp.float32)
            for e in range(1, E):
                eid = eid + jnp.where(tcl >= starts[:, e:e + 1], 1.0, 0.0)
            eid_ref[...] = eid.astype(jnp.int32)
            act_ref[...] = jnp.where(tt < total, 1, 0).astype(jnp.int32)


def _moe_kernel(eid_ref, act_ref, xs_ref, w1_ref, b1_ref, w2_ref, b2_ref,
                ys_ref):
    t = pl.program_id(0)

    @pl.when(act_ref[t] == 1)
    def _():
        h = jnp.maximum(
            jnp.dot(xs_ref[...], w1_ref[0], preferred_element_type=jnp.float32)
            + b1_ref[0],
            0.0,
        )
        ys_ref[...] = (
            jnp.dot(h, w2_ref[0], preferred_element_type=jnp.float32)
            + b2_ref[0]
        )


def _final_kernel(x_ref, c0_ref, c1_ref, gates_ref, g2_ref, b2_ref, o_ref):
    gates = gates_ref[...]
    moe = gates[:, 0:1] * c0_ref[...] + gates[:, 1:2] * c1_ref[...]
    o_ref[...] = _ln(x_ref[...] + moe, g2_ref[...], b2_ref[...])


def _dispatch_sc(x, pos0, pos1, cap):
    T, D = x.shape
    info = plsc.get_sparse_core_info()
    nw = info.num_cores * info.num_subcores
    ch = T // nw
    mesh = plsc.VectorSubcoreMesh(core_axis_name="c", subcore_axis_name="s")

    @functools.partial(
        pl.kernel, mesh=mesh,
        out_type=jax.ShapeDtypeStruct((cap, D), jnp.float32),
        scratch_types=[
            pltpu.VMEM((ch,), jnp.int32),
            pltpu.VMEM((ch,), jnp.int32),
            pltpu.VMEM((ch, D), jnp.float32),
            pltpu.SemaphoreType.DMA,
        ],
    )
    def k(x_hbm, p0_hbm, p1_hbm, xs_hbm, i0_v, i1_v, rows_v, sem):
        wid = lax.axis_index("s") * info.num_cores + lax.axis_index("c")
        base = wid * ch
        pltpu.sync_copy(x_hbm.at[pl.ds(base, ch)], rows_v)
        pltpu.sync_copy(p0_hbm.at[pl.ds(base, ch)], i0_v)
        pltpu.sync_copy(p1_hbm.at[pl.ds(base, ch)], i1_v)
        pltpu.async_copy(rows_v, xs_hbm.at[i0_v], sem).wait()
        pltpu.async_copy(rows_v, xs_hbm.at[i1_v], sem).wait()

    return k(x, pos0, pos1)


def _combine_sc(ys, pos0, pos1, T):
    D = ys.shape[1]
    info = plsc.get_sparse_core_info()
    nw = info.num_cores * info.num_subcores
    ch = T // nw
    mesh = plsc.VectorSubcoreMesh(core_axis_name="c", subcore_axis_name="s")

    @functools.partial(
        pl.kernel, mesh=mesh,
        out_type=[
            jax.ShapeDtypeStruct((T, D), jnp.float32),
            jax.ShapeDtypeStruct((T, D), jnp.float32),
        ],
        scratch_types=[
            pltpu.VMEM((ch,), jnp.int32),
            pltpu.VMEM((ch, D), jnp.float32),
            pltpu.SemaphoreType.DMA,
        ],
    )
    def k(ys_hbm, p0_hbm, p1_hbm, c0_hbm, c1_hbm, idx_v, rows_v, sem):
        wid = lax.axis_index("s") * info.num_cores + lax.axis_index("c")
        base = wid * ch
        pltpu.sync_copy(p0_hbm.at[pl.ds(base, ch)], idx_v)
        pltpu.async_copy(ys_hbm.at[idx_v], rows_v, sem).wait()
        pltpu.sync_copy(rows_v, c0_hbm.at[pl.ds(base, ch)])
        pltpu.sync_copy(p1_hbm.at[pl.ds(base, ch)], idx_v)
        pltpu.async_copy(ys_hbm.at[idx_v], rows_v, sem).wait()
        pltpu.sync_copy(rows_v, c1_hbm.at[pl.ds(base, ch)])

    return k(ys, pos0, pos1)


def kernel(src, in_proj_w, in_proj_b, out_proj_w, out_proj_b, ln1_g, ln1_b,
           ln2_g, ln2_b, router_w, router_b, w1, b1, w2, b2):
    Bq, T, D = src.shape
    dh = D // H
    F = w1.shape[2]
    x2 = src.reshape(T, D)

    nc = T // BC
    cap = 2 * T + (E - 1) * TILE
    nt = cap // TILE
    def xmap(i):
        return (jnp.maximum(i - 2 * nc, 0), 0)

    x, gates, pos0, pos1, eid, act = pl.pallas_call(
        functools.partial(_abc_kernel, T=T, scale=1.0 / (dh ** 0.5), dh=dh,
                          nt=nt, nc=nc),
        grid=(3 * nc,),
        in_specs=[
            pl.BlockSpec((BC, D), lambda i: (
                jnp.where(i < nc, i, jnp.maximum(i - 2 * nc, 0)), 0)),
            pl.BlockSpec((3 * D, D), lambda i: (0, 0)),
            pl.BlockSpec((1, 3 * D), lambda i: (0, 0)),
            pl.BlockSpec((D, D), lambda i: (0, 0)),
            pl.BlockSpec((1, D), lambda i: (0, 0)),
            pl.BlockSpec((1, D), lambda i: (0, 0)),
            pl.BlockSpec((1, D), lambda i: (0, 0)),
            pl.BlockSpec((D, E), lambda i: (0, 0)),
            pl.BlockSpec((1, E), lambda i: (0, 0)),
        ],
        out_specs=[
            pl.BlockSpec((BC, D), xmap),
            pl.BlockSpec((BC, 2), xmap),
            pl.BlockSpec((T, 1), lambda i: (0, 0)),
            pl.BlockSpec((T, 1), lambda i: (0, 0)),
            pl.BlockSpec((1, nt), lambda i: (0, 0)),
            pl.BlockSpec((1, nt), lambda i: (0, 0)),
        ],
        out_shape=[
            jax.ShapeDtypeStruct((T, D), jnp.float32),
            jax.ShapeDtypeStruct((T, 2), jnp.float32),
            jax.ShapeDtypeStruct((T, 1), jnp.int32),
            jax.ShapeDtypeStruct((T, 1), jnp.int32),
            jax.ShapeDtypeStruct((1, nt), jnp.int32),
            jax.ShapeDtypeStruct((1, nt), jnp.int32),
        ],
        scratch_shapes=[
            pltpu.VMEM((T, 3 * D), jnp.float32),
            pltpu.VMEM((T, D), jnp.float32),
            pltpu.VMEM((T, 2), jnp.int32),
            pltpu.VMEM((T, 2), jnp.int32),
            pltpu.VMEM((1, E), jnp.float32),
        ],
    )(x2, in_proj_w, in_proj_b.reshape(1, 3 * D),
      out_proj_w, out_proj_b.reshape(1, D),
      ln1_g.reshape(1, D), ln1_b.reshape(1, D),
      router_w, router_b.reshape(1, E))

    xs = _dispatch_sc(x, pos0.reshape(T), pos1.reshape(T), cap)

    grid_spec = pltpu.PrefetchScalarGridSpec(
        num_scalar_prefetch=2,
        grid=(nt,),
        in_specs=[
            pl.BlockSpec((TILE, D), lambda t, eid, act: (t, 0)),
            pl.BlockSpec((1, D, F), lambda t, eid, act: (eid[t], 0, 0)),
            pl.BlockSpec((1, 1, F), lambda t, eid, act: (eid[t], 0, 0)),
            pl.BlockSpec((1, F, D), lambda t, eid, act: (eid[t], 0, 0)),
            pl.BlockSpec((1, 1, D), lambda t, eid, act: (eid[t], 0, 0)),
        ],
        out_specs=pl.BlockSpec((TILE, D), lambda t, eid, act: (t, 0)),
    )
    ys = pl.pallas_call(
        _moe_kernel,
        grid_spec=grid_spec,
        out_shape=jax.ShapeDtypeStruct((cap, D), jnp.float32),
    )(eid.reshape(nt), act.reshape(nt), xs,
      w1, b1.reshape(E, 1, F),
      w2, b2.reshape(E, 1, D))

    c0, c1 = _combine_sc(ys, pos0.reshape(T), pos1.reshape(T), T)

    bf = min(512, T)
    out = pl.pallas_call(
        _final_kernel,
        grid=(T // bf,),
        in_specs=[
            pl.BlockSpec((bf, D), lambda i: (i, 0)),
            pl.BlockSpec((bf, D), lambda i: (i, 0)),
            pl.BlockSpec((bf, D), lambda i: (i, 0)),
            pl.BlockSpec((bf, 2), lambda i: (i, 0)),
            pl.BlockSpec((1, D), lambda i: (0, 0)),
            pl.BlockSpec((1, D), lambda i: (0, 0)),
        ],
        out_specs=pl.BlockSpec((bf, D), lambda i: (i, 0)),
        out_shape=jax.ShapeDtypeStruct((T, D), jnp.float32),
    )(x, c0, c1, gates, ln2_g.reshape(1, D), ln2_b.reshape(1, D))

    return out.reshape(Bq, T, D)
